# baseline (speedup 1.0000x reference)
import dataclasses
import functools

import jax
import jax.numpy as jnp
from jax import lax
from jax.experimental import pallas as pl
from jax.experimental.pallas import tpu as pltpu
from jax.experimental.pallas import tpu_sc as plsc

B, C, H, W = 4, 10, 512, 512
WPR = W // 32
NLANES = 16
SLABS = 8
ROWS_PER_SLAB = H // SLABS
MAXIT = H + W

_I32MIN = -2147483647 - 1


def _iota():
    return lax.iota(jnp.int32, NLANES)


def _lane_scalar(vec, lane):
    return jnp.max(jnp.where(_iota() == lane, vec, _I32MIN))


def _sc_flood(grid, scal):
    mesh = plsc.VectorSubcoreMesh(core_axis_name="c", subcore_axis_name="s",
                                  num_cores=2, num_subcores=16)
    cp = pltpu.CompilerParams()
    if "needs_layout_passes" in pltpu.CompilerParams.__dataclass_fields__:
        cp = dataclasses.replace(cp, needs_layout_passes=False)

    @functools.partial(
        pl.kernel,
        mesh=mesh,
        out_type=[
            jax.ShapeDtypeStruct((B, H * WPR), jnp.int32),
            jax.ShapeDtypeStruct((B, NLANES), jnp.int32),
        ],
        scratch_types=[
            pltpu.VMEM((ROWS_PER_SLAB, W), jnp.float32),
            pltpu.VMEM((ROWS_PER_SLAB * WPR,), jnp.int32),
            pltpu.VMEM((H * WPR,), jnp.int32),
            pltpu.VMEM((H * WPR,), jnp.int32),
            pltpu.VMEM((NLANES, NLANES), jnp.float32),
            pltpu.VMEM((NLANES,), jnp.int32),
            pltpu.VMEM((NLANES,), jnp.int32),
            pltpu.VMEM_SHARED((2, H * WPR), jnp.int32),
            pltpu.SemaphoreType.DMA,
        ],
        compiler_params=cp,
    )
    def k(grid_hbm, scal_hbm, mask_hbm, info_hbm,
          rowbuf, packslice, region, maskw, valbuf, scalbuf, infovec,
          packed_sh, dsem):
        core = lax.axis_index("c")
        sub = lax.axis_index("s")
        bic = sub // SLABS
        b = core * 2 + bic
        slab = sub % SLABS
        iota = _iota()
        zero16 = jnp.zeros((NLANES,), jnp.int32)

        pltpu.sync_copy(scal_hbm, scalbuf)
        sv = scalbuf[...]
        seed_y = _lane_scalar(sv, 0)
        seed_x = _lane_scalar(sv, 1)
        tgt_ch = _lane_scalar(sv, 2)

        handles = [
            pltpu.async_copy(grid_hbm.at[b, cc, seed_y, pl.ds(seed_x, 1)],
                             valbuf.at[cc, pl.ds(0, 1)], dsem)
            for cc in range(C)
        ]
        for h in handles:
            h.wait()
        vals = plsc.load_gather(
            valbuf, [jnp.minimum(iota, C - 1), iota * 0])
        has = (vals > 0.5) & (iota < C)
        valid = jnp.max(plsc.all_reduce_population_count(has)) > 0
        ffs = jnp.max(plsc.all_reduce_ffs(has))
        seed_ch = jnp.where(valid, jnp.minimum(ffs, C - 1), 0)

        r0 = slab * ROWS_PER_SLAB
        pltpu.sync_copy(
            grid_hbm.at[b, seed_ch, pl.ds(r0, ROWS_PER_SLAB), :], rowbuf)

        col0 = iota * 32

        @pl.loop(0, ROWS_PER_SLAB)
        def _pack_row(i):
            irow = jnp.full((NLANES,), i, jnp.int32)
            acc = zero16
            for kk in range(32):
                v = plsc.load_gather(rowbuf, [irow, col0 + kk])
                bitc = (1 << kk) if kk < 31 else _I32MIN
                acc = acc | jnp.where(v > 0.5, jnp.int32(bitc), 0)
            packslice[pl.ds(i * WPR, NLANES)] = acc

        pltpu.sync_copy(packslice,
                        packed_sh.at[bic, pl.ds(r0 * WPR, ROWS_PER_SLAB * WPR)])
        plsc.subcore_barrier()

        is_fill_tile = slab == 0

        @pl.when(is_fill_tile)
        def _fill():
            pltpu.sync_copy(packed_sh.at[bic], region)

            @pl.loop(0, H)
            def _zrow(r):
                maskw[pl.ds(r * WPR, NLANES)] = zero16

            lane0 = lax.shift_right_logical(seed_x, 5)
            bit0 = seed_x & 31
            regrow = region[pl.ds(seed_y * WPR, NLANES)]
            init = jnp.where(
                (iota == lane0) & valid,
                lax.shift_left(jnp.int32(1), bit0), 0) & regrow
            maskw[pl.ds(seed_y * WPR, NLANES)] = init
            init_changed = jnp.any(init != 0)

            def horiz(v, reg):
                return (v
                        | lax.shift_left(v, 1)
                        | lax.shift_right_logical(v, 1)) & reg

            def step_cond(carry):
                ylo, yhi, it, changed = carry
                return changed & (it < MAXIT)

            def update_row(r, prev):
                base = r * WPR
                m = maskw[pl.ds(base, NLANES)]
                reg = region[pl.ds(base, NLANES)]
                xl = plsc.load_gather(
                    maskw, [base + jnp.maximum(iota - 1, 0)])
                xl = jnp.where(iota >= 1, xl, 0)
                xr = plsc.load_gather(
                    maskw, [base + jnp.minimum(iota + 1, NLANES - 1)])
                xr = jnp.where(iota <= NLANES - 2, xr, 0)
                new = (m
                       | lax.shift_left(m, 1)
                       | lax.shift_right_logical(m, 1)
                       | lax.shift_right_logical(xl, 31)
                       | lax.shift_left(xr, 31)
                       | prev) & reg
                return m, new, reg, base

            def gs_sweep(carry):
                ylo, yhi, it, _ = carry
                down = (it & 1) == 0
                start = jnp.where(down, jnp.maximum(ylo - 1, 0),
                                  jnp.minimum(yhi + 1, H - 1))
                end0 = jnp.where(down, jnp.minimum(yhi + 1, H - 1),
                                 jnp.maximum(ylo - 1, 0))
                stp = jnp.where(down, 1, -1)

                def sw_cond(c):
                    r, end, prev, chg, ymin, ymax = c
                    return jnp.where(down, r <= end, r >= end)

                def sw_body(c):
                    r, end, prev, chg, ymin, ymax = c
                    a_idx = jnp.clip(r + stp, 0, H - 1)
                    ahead = maskw[pl.ds(a_idx * WPR, NLANES)]
                    ahead = jnp.where(
                        jnp.where(down, r < H - 1, r > 0), ahead, 0)
                    m, new, reg, base = update_row(r, prev | ahead)
                    maskw[pl.ds(base, NLANES)] = new
                    xl2 = plsc.load_gather(
                        maskw, [base + jnp.maximum(iota - 1, 0)])
                    xl2 = jnp.where(iota >= 1, xl2, 0)
                    xr2 = plsc.load_gather(
                        maskw, [base + jnp.minimum(iota + 1, NLANES - 1)])
                    xr2 = jnp.where(iota <= NLANES - 2, xr2, 0)
                    v = (new
                         | lax.shift_right_logical(xl2, 31)
                         | lax.shift_left(xr2, 31)) & reg
                    new = horiz(horiz(v, reg), reg)
                    maskw[pl.ds(base, NLANES)] = new
                    chg = chg | (new ^ m)
                    nz = jnp.any(new != 0)
                    ymin = jnp.where(nz, jnp.minimum(ymin, r), ymin)
                    ymax = jnp.where(nz, jnp.maximum(ymax, r), ymax)
                    can_ext = jnp.where(down, end < H - 1, end > 0)
                    end = jnp.where((r == end) & nz & can_ext,
                                    end + stp, end)
                    return r + stp, end, new, chg, ymin, ymax

                _, _, _, chg, ymin, ymax = lax.while_loop(
                    sw_cond, sw_body,
                    (start, end0, zero16, zero16, seed_y, seed_y))
                changed = jnp.any(chg != 0)
                return (jnp.minimum(ymin, ylo), jnp.maximum(ymax, yhi),
                        it + 1, changed)

            def sync_step(carry):
                ylo, yhi, it, _ = carry
                lo = jnp.maximum(ylo - 1, 0)
                hi = jnp.minimum(yhi + 1, H - 1)

                def row_body(r, rc):
                    prev_old, chg, lorow, hirow = rc
                    nxt = maskw[pl.ds(jnp.minimum(r + 1, H - 1) * WPR,
                                      NLANES)]
                    nxt = jnp.where(r < H - 1, nxt, 0)
                    m, new, reg, base = update_row(r, prev_old | nxt)
                    maskw[pl.ds(base, NLANES)] = new
                    chg = chg | (new ^ m)
                    lorow = jnp.where(r == lo, new, lorow)
                    hirow = jnp.where(r == hi, new, hirow)
                    return m, chg, lorow, hirow

                _, chg, lorow, hirow = lax.fori_loop(
                    lo, hi + 1, row_body, (zero16, zero16, zero16, zero16))
                changed = jnp.any(chg != 0)
                ylo2 = jnp.where(jnp.any(lorow != 0), lo, ylo)
                yhi2 = jnp.where(jnp.any(hirow != 0), hi, yhi)
                return ylo2, yhi2, it + 1, changed

            ylo, yhi, _, gs_changed = lax.while_loop(
                step_cond, gs_sweep,
                (seed_y, seed_y, jnp.int32(0), init_changed))

            def pop_body(r, acc):
                m = maskw[pl.ds(r * WPR, NLANES)]
                x = m - (lax.shift_right_logical(m, 1) & 0x55555555)
                x = ((x & 0x33333333)
                     + (lax.shift_right_logical(x, 2) & 0x33333333))
                x = (x + lax.shift_right_logical(x, 4)) & 0x0F0F0F0F
                return acc + lax.shift_right_logical(x * 0x01010101, 24)

            popv = lax.fori_loop(ylo, yhi + 1, pop_body, zero16)
            pop_total = jnp.sum(popv)
            exact = (~gs_changed) & (pop_total <= MAXIT)

            @pl.when(~exact)
            def _fallback_sync():
                @pl.loop(0, H)
                def _zrow2(r):
                    maskw[pl.ds(r * WPR, NLANES)] = zero16
                maskw[pl.ds(seed_y * WPR, NLANES)] = init
                lax.while_loop(
                    step_cond, sync_step,
                    (seed_y, seed_y, jnp.int32(0), init_changed))

            pltpu.sync_copy(maskw, mask_hbm.at[b])
            infovec[...] = (jnp.where(iota == 0, seed_ch, 0)
                            | jnp.where(iota == 1, tgt_ch, 0))
            pltpu.sync_copy(infovec, info_hbm.at[b])

    return k(grid, scal)


def _tc_fix_body(info_ref, g_ref, pm_ref, o_ref, mbuf):
    j = pl.program_id(1)

    @pl.when(j == 0)
    def _expand():
        pw = pm_ref[0]
        lo = (pw & 0xFFFF).astype(jnp.float32)
        hi = lax.shift_right_logical(pw, 16).astype(jnp.float32)
        wq = lax.broadcasted_iota(jnp.int32, (WPR, W), 0)
        xq = lax.broadcasted_iota(jnp.int32, (WPR, W), 1)
        onehot = (lax.shift_right_logical(xq, 5) == wq).astype(jnp.float32)
        slo = jnp.dot(lo, onehot, preferred_element_type=jnp.float32,
                      precision=lax.Precision.HIGHEST)
        shi = jnp.dot(hi, onehot, preferred_element_type=jnp.float32,
                      precision=lax.Precision.HIGHEST)
        xb = lax.broadcasted_iota(jnp.int32, (H, W), 1) & 31
        word = jnp.where(xb >= 16, shi.astype(jnp.int32),
                         slo.astype(jnp.int32))
        bit = lax.shift_right_logical(word, xb & 15) & 1
        mbuf[...] = bit.astype(jnp.float32)

    m = mbuf[...]
    val = jnp.where(j == 1, 1.0, 0.0)
    g = g_ref[0, 0]
    o_ref[0, 0] = g * (1.0 - m) + m * val


def _tc_fix(grid, maskp, info):
    grid_spec = pltpu.PrefetchScalarGridSpec(
        num_scalar_prefetch=1,
        grid=(B, 2),
        in_specs=[
            pl.BlockSpec((1, 1, H, W),
                         lambda b, j, info: (b, info[b, j], 0, 0)),
            pl.BlockSpec((1, H, WPR), lambda b, j, info: (b, 0, 0)),
        ],
        out_specs=pl.BlockSpec((1, 1, H, W),
                               lambda b, j, info: (b, info[b, j], 0, 0)),
        scratch_shapes=[pltpu.VMEM((H, W), jnp.float32)],
    )
    return pl.pallas_call(
        _tc_fix_body,
        grid_spec=grid_spec,
        out_shape=jax.ShapeDtypeStruct((B, C, H, W), jnp.float32),
        input_output_aliases={1: 0},
    )(info, grid, maskp)


def kernel(grid, seed_y, seed_x, target_color):
    sy = jnp.asarray(seed_y, jnp.int32)
    sx = jnp.asarray(seed_x, jnp.int32)
    tc = jnp.asarray(target_color, jnp.int32)
    scal = (jnp.zeros((NLANES,), jnp.int32)
            .at[0].set(sy).at[1].set(sx).at[2].set(tc))
    maskp, info = _sc_flood(grid, scal)
    return _tc_fix(grid, maskp.reshape(B, H, WPR), info)


# speedup vs baseline: 1.1023x; 1.1023x over previous
import dataclasses
import functools

import jax
import jax.numpy as jnp
from jax import lax
from jax.experimental import pallas as pl
from jax.experimental.pallas import tpu as pltpu
from jax.experimental.pallas import tpu_sc as plsc

B, C, H, W = 4, 10, 512, 512
WPR = W // 32
NLANES = 16
SLABS = 8
ROWS_PER_SLAB = H // SLABS
MAXIT = H + W

_I32MIN = -2147483647 - 1


def _iota():
    return lax.iota(jnp.int32, NLANES)


def _lane_scalar(vec, lane):
    return jnp.max(jnp.where(_iota() == lane, vec, _I32MIN))


def _sc_flood(grid, scal):
    mesh = plsc.VectorSubcoreMesh(core_axis_name="c", subcore_axis_name="s",
                                  num_cores=2, num_subcores=16)
    cp = pltpu.CompilerParams()
    if "needs_layout_passes" in pltpu.CompilerParams.__dataclass_fields__:
        cp = dataclasses.replace(cp, needs_layout_passes=False)

    @functools.partial(
        pl.kernel,
        mesh=mesh,
        out_type=[
            jax.ShapeDtypeStruct((B, H * WPR), jnp.int32),
            jax.ShapeDtypeStruct((B, NLANES), jnp.int32),
        ],
        scratch_types=[
            pltpu.VMEM((ROWS_PER_SLAB, W), jnp.float32),
            pltpu.VMEM((ROWS_PER_SLAB * WPR,), jnp.int32),
            pltpu.VMEM((H * WPR,), jnp.int32),
            pltpu.VMEM((H * WPR,), jnp.int32),
            pltpu.VMEM((NLANES, NLANES), jnp.float32),
            pltpu.VMEM((NLANES,), jnp.int32),
            pltpu.VMEM((NLANES,), jnp.int32),
            pltpu.VMEM_SHARED((2, H * WPR), jnp.int32),
            pltpu.VMEM_SHARED((2, NLANES), jnp.int32),
            pltpu.SemaphoreType.DMA,
        ],
        compiler_params=cp,
    )
    def k(grid_hbm, scal_hbm, mask_hbm, info_hbm,
          rowbuf, packslice, region, maskw, valbuf, scalbuf, infovec,
          packed_sh, seed_sh, dsem):
        core = lax.axis_index("c")
        sub = lax.axis_index("s")
        bic = sub // SLABS
        b = core * 2 + bic
        slab = sub % SLABS
        iota = _iota()
        zero16 = jnp.zeros((NLANES,), jnp.int32)

        pltpu.sync_copy(scal_hbm, scalbuf)
        sv = scalbuf[...]
        seed_y = _lane_scalar(sv, 0)
        seed_x = _lane_scalar(sv, 1)
        tgt_ch = _lane_scalar(sv, 2)

        is_fill_tile = slab == 0

        @pl.when(is_fill_tile)
        def _seed_read():
            handles = [
                pltpu.async_copy(grid_hbm.at[b, cc, seed_y, pl.ds(seed_x, 1)],
                                 valbuf.at[cc, pl.ds(0, 1)], dsem)
                for cc in range(C)
            ]
            for h in handles:
                h.wait()
            vals = plsc.load_gather(
                valbuf, [jnp.minimum(iota, C - 1), iota * 0])
            has = (vals > 0.5) & (iota < C)
            valid0 = jnp.max(plsc.all_reduce_population_count(has)) > 0
            ffs = jnp.max(plsc.all_reduce_ffs(has))
            sc0 = jnp.where(valid0, jnp.minimum(ffs, C - 1), 0)
            infovec[...] = (jnp.where(iota == 0, sc0, 0)
                            | jnp.where(iota == 1,
                                        jnp.where(valid0, 1, 0), 0))
            pltpu.sync_copy(infovec, seed_sh.at[bic])

        plsc.subcore_barrier()
        pltpu.sync_copy(seed_sh.at[bic], scalbuf)
        sv2 = scalbuf[...]
        seed_ch = _lane_scalar(sv2, 0)
        valid = _lane_scalar(sv2, 1) > 0

        r0 = slab * ROWS_PER_SLAB
        pltpu.sync_copy(
            grid_hbm.at[b, seed_ch, pl.ds(r0, ROWS_PER_SLAB), :], rowbuf)

        col0 = iota * 32

        @pl.loop(0, ROWS_PER_SLAB)
        def _pack_row(i):
            irow = jnp.full((NLANES,), i, jnp.int32)
            acc = zero16
            for kk in range(32):
                v = plsc.load_gather(rowbuf, [irow, col0 + kk])
                bitc = (1 << kk) if kk < 31 else _I32MIN
                acc = acc | jnp.where(v > 0.5, jnp.int32(bitc), 0)
            packslice[pl.ds(i * WPR, NLANES)] = acc

        pltpu.sync_copy(packslice,
                        packed_sh.at[bic, pl.ds(r0 * WPR, ROWS_PER_SLAB * WPR)])
        plsc.subcore_barrier()

        @pl.when(is_fill_tile)
        def _fill():
            pltpu.sync_copy(packed_sh.at[bic], region)

            @pl.loop(0, H // 8)
            def _zrow(r8):
                for dr in range(8):
                    maskw[pl.ds((r8 * 8 + dr) * WPR, NLANES)] = zero16

            lane0 = lax.shift_right_logical(seed_x, 5)
            bit0 = seed_x & 31
            regrow = region[pl.ds(seed_y * WPR, NLANES)]
            init = jnp.where(
                (iota == lane0) & valid,
                lax.shift_left(jnp.int32(1), bit0), 0) & regrow
            maskw[pl.ds(seed_y * WPR, NLANES)] = init
            init_changed = jnp.any(init != 0)

            def step_cond(carry):
                ylo, yhi, it, changed = carry
                return changed & (it < MAXIT)

            def update_row(r, prev):
                base = r * WPR
                m = maskw[pl.ds(base, NLANES)]
                reg = region[pl.ds(base, NLANES)]
                xl = plsc.load_gather(
                    maskw, [base + jnp.maximum(iota - 1, 0)])
                xl = jnp.where(iota >= 1, xl, 0)
                xr = plsc.load_gather(
                    maskw, [base + jnp.minimum(iota + 1, NLANES - 1)])
                xr = jnp.where(iota <= NLANES - 2, xr, 0)
                new = (m
                       | lax.shift_left(m, 1)
                       | lax.shift_right_logical(m, 1)
                       | lax.shift_right_logical(xl, 31)
                       | lax.shift_left(xr, 31)
                       | prev) & reg
                return m, new, reg, base

            def sync_step(carry):
                ylo, yhi, it, _ = carry
                lo = jnp.maximum(ylo - 1, 0)
                hi = jnp.minimum(yhi + 1, H - 1)

                def row_body(r, rc):
                    prev_old, chg, lorow, hirow = rc
                    nxt = maskw[pl.ds(jnp.minimum(r + 1, H - 1) * WPR,
                                      NLANES)]
                    nxt = jnp.where(r < H - 1, nxt, 0)
                    m, new, reg, base = update_row(r, prev_old | nxt)
                    maskw[pl.ds(base, NLANES)] = new
                    chg = chg | (new ^ m)
                    lorow = jnp.where(r == lo, new, lorow)
                    hirow = jnp.where(r == hi, new, hirow)
                    return m, chg, lorow, hirow

                _, chg, lorow, hirow = lax.fori_loop(
                    lo, hi + 1, row_body, (zero16, zero16, zero16, zero16))
                changed = jnp.any(chg != 0)
                ylo2 = jnp.where(jnp.any(lorow != 0), lo, ylo)
                yhi2 = jnp.where(jnp.any(hirow != 0), hi, yhi)
                return ylo2, yhi2, it + 1, changed

            lax.while_loop(
                step_cond, sync_step,
                (seed_y, seed_y, jnp.int32(0), init_changed))

            pltpu.sync_copy(maskw, mask_hbm.at[b])
            infovec[...] = (jnp.where(iota == 0, seed_ch, 0)
                            | jnp.where(iota == 1, tgt_ch, 0))
            pltpu.sync_copy(infovec, info_hbm.at[b])

    return k(grid, scal)


def _tc_fix_body(info_ref, g_ref, pm_ref, o_ref, mbuf):
    j = pl.program_id(1)

    @pl.when(j == 0)
    def _expand():
        pw = pm_ref[0]
        wq = lax.broadcasted_iota(jnp.int32, (WPR, W), 0)
        xq = lax.broadcasted_iota(jnp.int32, (WPR, W), 1)
        onehot = (lax.shift_right_logical(xq, 5) == wq).astype(jnp.bfloat16)
        planes = []
        for sh in (0, 8, 16, 24):
            cb = ((lax.shift_right_logical(pw, sh) & 0xFF)
                  .astype(jnp.float32).astype(jnp.bfloat16))
            planes.append(
                jnp.dot(cb, onehot, preferred_element_type=jnp.float32)
                .astype(jnp.int32))
        xb = lax.broadcasted_iota(jnp.int32, (H, W), 1) & 31
        bsel = lax.shift_right_logical(xb, 3)
        byte = jnp.where(
            bsel >= 2,
            jnp.where(bsel == 3, planes[3], planes[2]),
            jnp.where(bsel == 1, planes[1], planes[0]))
        bit = lax.shift_right_logical(byte, xb & 7) & 1
        mbuf[...] = bit.astype(jnp.float32)

    m = mbuf[...]
    val = jnp.where(j == 1, 1.0, 0.0)
    g = g_ref[0, 0]
    o_ref[0, 0] = g * (1.0 - m) + m * val


def _tc_fix(grid, maskp, info):
    grid_spec = pltpu.PrefetchScalarGridSpec(
        num_scalar_prefetch=1,
        grid=(B, 2),
        in_specs=[
            pl.BlockSpec((1, 1, H, W),
                         lambda b, j, info: (b, info[b, j], 0, 0)),
            pl.BlockSpec((1, H, WPR), lambda b, j, info: (b, 0, 0)),
        ],
        out_specs=pl.BlockSpec((1, 1, H, W),
                               lambda b, j, info: (b, info[b, j], 0, 0)),
        scratch_shapes=[pltpu.VMEM((H, W), jnp.float32)],
    )
    return pl.pallas_call(
        _tc_fix_body,
        grid_spec=grid_spec,
        out_shape=jax.ShapeDtypeStruct((B, C, H, W), jnp.float32),
        input_output_aliases={1: 0},
    )(info, grid, maskp)


def kernel(grid, seed_y, seed_x, target_color):
    sy = jnp.asarray(seed_y, jnp.int32)
    sx = jnp.asarray(seed_x, jnp.int32)
    tc = jnp.asarray(target_color, jnp.int32)
    scal = (jnp.zeros((NLANES,), jnp.int32)
            .at[0].set(sy).at[1].set(sx).at[2].set(tc))
    maskp, info = _sc_flood(grid, scal)
    return _tc_fix(grid, maskp.reshape(B, H, WPR), info)
